# baseline (speedup 1.0000x reference)
import jax
import jax.numpy as jnp
from jax import lax
from jax.experimental import pallas as pl
from jax.experimental.pallas import tpu as pltpu
from jax.experimental.pallas import tpu_sc as plsc

N = 10000
D = 128
G = 16
NLAYERS = 6

NC = 2
NS = 16
NW = NC * NS

K = 128
CH0 = 57
CH1 = 105
CHMAX = max(CH0, CH1)
NCH0T = NS * CH0
NCH = NS * (CH0 + CH1)
E_PAD = NCH * K
E_REAL = 320000 + N
PAD = E_PAD - E_REAL

NROW = 10240
SCRAP = N
STRIPE = NROW // NS
NBUF = 4
D2 = D // 2

_mesh = plsc.VectorSubcoreMesh(
    core_axis_name="c", subcore_axis_name="s", num_cores=NC, num_subcores=NS)

_f32 = jnp.float32


def _tile_chunks(c, s):
    ch_n = jnp.where(c == 0, CH0, CH1)
    off = jnp.where(c == 0, s * CH0, NCH0T + s * CH1)
    return ch_n, off


def _sc_deg_body(dst_hbm, out_hbm, didx, deg):
    c = lax.axis_index("c")
    s = lax.axis_index("s")
    wid = c * NS + s
    ch_n, off = _tile_chunks(c, s)
    pltpu.sync_copy(dst_hbm.at[pl.ds(off, CHMAX)], didx)
    zero = jnp.zeros((16,), _f32)
    ones = jnp.ones((16,), _f32)

    @pl.loop(0, NROW // 16)
    def _z(i):
        deg[pl.ds(i * 16, 16)] = zero

    @pl.loop(0, ch_n)
    def _ch(j):
        @pl.loop(0, K // 16)
        def _v(k):
            idx = didx[j, pl.ds(k * 16, 16)]
            plsc.addupdate_scatter(deg, [idx], ones)

    pltpu.sync_copy(deg, out_hbm.at[wid])


_sc_deg = pl.kernel(
    _sc_deg_body,
    out_type=jax.ShapeDtypeStruct((NW, NROW), _f32),
    mesh=_mesh,
    compiler_params=pltpu.CompilerParams(
        needs_layout_passes=False, use_tc_tiling_on_sc=False),
    scratch_types=[
        pltpu.VMEM((CHMAX, K), jnp.int32),
        pltpu.VMEM((NROW,), _f32),
    ],
)


def _sc_msg_body(u0_hbm, u1_hbm, src_hbm, dst_hbm, out_hbm,
                 sidx, didx, rows, zbuf, acc, gsem, ssem):
    c = lax.axis_index("c")
    s = lax.axis_index("s")
    ch_n, off = _tile_chunks(c, s)
    pltpu.sync_copy(src_hbm.at[pl.ds(off, CHMAX)], sidx)
    pltpu.sync_copy(dst_hbm.at[pl.ds(off, CHMAX)], didx)

    zero = jnp.zeros((16,), _f32)

    @pl.loop(0, K)
    def _zrow(i):
        @pl.loop(0, D2 // 16)
        def _zcol(j):
            zbuf[i, pl.ds(j * 16, 16)] = zero

    def _zero_stripe():
        @pl.loop(0, STRIPE // K)
        def _zacc(t):
            pltpu.sync_copy(zbuf, acc.at[pl.ds(s * STRIPE + t * K, K)])

    def _prologue(utab):
        for j in range(NBUF - 1):
            pltpu.async_copy(utab.at[sidx.at[j]], rows.at[j], gsem)

    def _phase(utab):
        @pl.loop(0, ch_n)
        def _edge(j):
            cur = lax.rem(j, NBUF)
            pltpu.make_async_copy(utab.at[sidx.at[j]], rows.at[cur], gsem).wait()

            @pl.when(j + NBUF - 1 < ch_n)
            def _prefetch():
                pltpu.async_copy(utab.at[sidx.at[j + NBUF - 1]],
                                 rows.at[lax.rem(j + NBUF - 1, NBUF)], gsem)

            pltpu.sync_copy(rows.at[cur], acc.at[didx.at[j]], add=True)

    _prologue(u0_hbm)
    _zero_stripe()
    plsc.subcore_barrier()
    _phase(u0_hbm)
    _prologue(u1_hbm)
    plsc.subcore_barrier()
    pltpu.sync_copy(acc.at[pl.ds(s * STRIPE, STRIPE)],
                    out_hbm.at[c, 0, pl.ds(s * STRIPE, STRIPE)])
    _zero_stripe()
    plsc.subcore_barrier()
    _phase(u1_hbm)
    plsc.subcore_barrier()
    pltpu.sync_copy(acc.at[pl.ds(s * STRIPE, STRIPE)],
                    out_hbm.at[c, 1, pl.ds(s * STRIPE, STRIPE)])


_sc_msg = pl.kernel(
    _sc_msg_body,
    out_type=jax.ShapeDtypeStruct((NC, 2, NROW, D2), _f32),
    mesh=_mesh,
    compiler_params=pltpu.CompilerParams(
        needs_layout_passes=False, use_tc_tiling_on_sc=False),
    scratch_types=[
        pltpu.VMEM((CHMAX, K), jnp.int32),
        pltpu.VMEM((CHMAX, K), jnp.int32),
        pltpu.VMEM((NBUF, K, D2), _f32),
        pltpu.VMEM((K, D2), _f32),
        pltpu.VMEM_SHARED((NROW, D2), _f32),
        pltpu.SemaphoreType.DMA,
        pltpu.SemaphoreType.DMA,
    ],
)


def _bn_tc(x, g, b, eps=1e-5):
    mu = jnp.mean(x, axis=0, keepdims=True)
    xc = x - mu
    var = jnp.mean(xc * xc, axis=0, keepdims=True)
    return xc * lax.rsqrt(var + eps) * g + b


def _matmul_wt(h, w):
    return lax.dot_general(h, w, (((1,), (1,)), ((), ())),
                           preferred_element_type=_f32)


def _dinv_body(parts_ref, out_ref):
    deg = jnp.sum(parts_ref[...], axis=0, keepdims=True)
    deg_c = jnp.maximum(deg, 1e-12)
    out_ref[...] = jnp.where(deg > 0, lax.rsqrt(deg_c), 0.0)


def _split_cols(u, out_ref):
    out_ref[0, :, :] = u[:, :D2]
    out_ref[1, :, :] = u[:, D2:]


def _merge_s(s_ref):
    return jnp.concatenate(
        [s_ref[0, 0, :N, :] + s_ref[1, 0, :N, :],
         s_ref[0, 1, :N, :] + s_ref[1, 1, :N, :]], axis=1)


def _pre_body(x_ref, w_ref, ewr, ebr, g0, b0, gw, bw, w0, dinv_ref, out_ref):
    h = _bn_tc(x_ref[...], g0[...], b0[...])
    ew = w_ref[...] * ewr[...] + ebr[...]
    ew = _bn_tc(ew, gw[...], bw[...])
    h = jnp.maximum(h + ew, 0.0)
    u = _matmul_wt(h, w0[...])
    _split_cols(u * dinv_ref[...][:N, :], out_ref)


def _mid_body(s_ref, dinv_ref, bprev, g, bb, w, out_ref):
    dinv = dinv_ref[...][:N, :]
    agg = _merge_s(s_ref) * dinv + bprev[...]
    h = jnp.maximum(_bn_tc(agg, g[...], bb[...]), 0.0)
    _split_cols(_matmul_wt(h, w[...]) * dinv, out_ref)


def _final_body(s_ref, dinv_ref, b5, batch_ref, lw, lb, out_ref):
    dinv = dinv_ref[...][:N, :]
    agg = _merge_s(s_ref) * dinv + b5[...]
    onehot = (batch_ref[...] == lax.broadcasted_iota(
        jnp.int32, (1, G), 1)).astype(_f32)
    sums = lax.dot_general(onehot, agg, (((0,), (0,)), ((), ())),
                           preferred_element_type=_f32)
    cnt = lax.dot_general(onehot, jnp.ones((N, 1), _f32),
                          (((0,), (0,)), ((), ())),
                          preferred_element_type=_f32)
    red = jnp.sum(sums * lw[...], axis=1, keepdims=True)
    out_ref[...] = red / jnp.maximum(cnt, 1.0) + lb[0, 0]


def _tc(body, out_shape, *args):
    return pl.pallas_call(
        body, out_shape=jax.ShapeDtypeStruct(out_shape, _f32))(*args)


def kernel(x, edge_index, batch, weights, embed_W, embed_b, bn0_g, bn0_b,
           bnw_g, bnw_b, conv_W, conv_b, bn_g, bn_b, lin_W, lin_b):
    i32 = jnp.int32
    loop_idx = jnp.arange(N, dtype=i32)
    src = jnp.concatenate(
        [edge_index[0].astype(i32), loop_idx,
         jnp.zeros((PAD + CHMAX * K,), i32)]).reshape(NCH + CHMAX, K)
    dst = jnp.concatenate(
        [edge_index[1].astype(i32), loop_idx,
         jnp.full((PAD + CHMAX * K,), SCRAP, i32)]).reshape(NCH + CHMAX, K)

    deg_parts = _sc_deg(dst)
    dinv_col = _tc(_dinv_body, (1, NROW), deg_parts).reshape(NROW, 1)

    row = lambda v: v.reshape(1, D).astype(_f32)
    u = _tc(_pre_body, (2, N, D2),
            x.astype(_f32), weights.reshape(N, 1).astype(_f32),
            row(embed_W), row(embed_b), row(bn0_g), row(bn0_b),
            row(bnw_g), row(bnw_b), conv_W[0], dinv_col)

    for i in range(1, NLAYERS):
        sparts = _sc_msg(u[0], u[1], src, dst)
        u = _tc(_mid_body, (2, N, D2),
                sparts, dinv_col, row(conv_b[i - 1]),
                row(bn_g[i - 1]), row(bn_b[i - 1]), conv_W[i])

    sparts = _sc_msg(u[0], u[1], src, dst)
    out = _tc(_final_body, (G, 1),
              sparts, dinv_col, row(conv_b[NLAYERS - 1]),
              batch.reshape(N, 1).astype(i32), lin_W.astype(_f32),
              lin_b.reshape(1, 1).astype(_f32))
    return out.reshape(G)


# speedup vs baseline: 1.1691x; 1.1691x over previous
import jax
import jax.numpy as jnp
from jax import lax
from jax.experimental import pallas as pl
from jax.experimental.pallas import tpu as pltpu
from jax.experimental.pallas import tpu_sc as plsc

N = 10000
D = 128
G = 16
NLAYERS = 6

NC = 2
NS = 16
NW = NC * NS

K = 128
CH0 = 105
CH1 = 57
CHMAX = max(CH0, CH1)
NCH0T = NS * CH0
NCH = NS * (CH0 + CH1)
E_PAD = NCH * K
E_REAL = 320000 + N
PAD = E_PAD - E_REAL

NROW = 10240
SCRAP = N
STRIPE = NROW // NS
NBUF = 4
D2 = D // 2

_mesh = plsc.VectorSubcoreMesh(
    core_axis_name="c", subcore_axis_name="s", num_cores=NC, num_subcores=NS)

_f32 = jnp.float32


def _tile_chunks(c, s):
    ch_n = jnp.where(c == 0, CH0, CH1)
    off = jnp.where(c == 0, s * CH0, NCH0T + s * CH1)
    return ch_n, off


def _sc_deg_body(dst_hbm, out_hbm, didx, deg):
    c = lax.axis_index("c")
    s = lax.axis_index("s")
    wid = c * NS + s
    ch_n, off = _tile_chunks(c, s)
    pltpu.sync_copy(dst_hbm.at[pl.ds(off, CHMAX)], didx)
    zero = jnp.zeros((16,), _f32)
    ones = jnp.ones((16,), _f32)

    @pl.loop(0, NROW // 16)
    def _z(i):
        deg[pl.ds(i * 16, 16)] = zero

    @pl.loop(0, ch_n)
    def _ch(j):
        @pl.loop(0, K // 16)
        def _v(k):
            idx = didx[j, pl.ds(k * 16, 16)]
            plsc.addupdate_scatter(deg, [idx], ones)

    pltpu.sync_copy(deg, out_hbm.at[wid])


_sc_deg = pl.kernel(
    _sc_deg_body,
    out_type=jax.ShapeDtypeStruct((NW, NROW), _f32),
    mesh=_mesh,
    compiler_params=pltpu.CompilerParams(
        needs_layout_passes=False, use_tc_tiling_on_sc=False),
    scratch_types=[
        pltpu.VMEM((CHMAX, K), jnp.int32),
        pltpu.VMEM((NROW,), _f32),
    ],
)


def _sc_msg_body(u0_hbm, u1_hbm, src_hbm, dst_hbm, out_hbm,
                 sidx, didx, rows, zbuf, acc, gsem, ssem):
    c = lax.axis_index("c")
    s = lax.axis_index("s")
    ch_n, off = _tile_chunks(c, s)
    pltpu.sync_copy(src_hbm.at[pl.ds(off, CHMAX)], sidx)
    pltpu.sync_copy(dst_hbm.at[pl.ds(off, CHMAX)], didx)

    zero = jnp.zeros((16,), _f32)

    @pl.loop(0, K)
    def _zrow(i):
        @pl.loop(0, D2 // 16)
        def _zcol(j):
            zbuf[i, pl.ds(j * 16, 16)] = zero

    def _zero_stripe():
        @pl.loop(0, STRIPE // K)
        def _zacc(t):
            pltpu.sync_copy(zbuf, acc.at[pl.ds(s * STRIPE + t * K, K)])

    def _prologue(utab):
        for j in range(NBUF - 1):
            pltpu.async_copy(utab.at[sidx.at[j]], rows.at[j], gsem)

    def _phase(utab):
        @pl.loop(0, ch_n)
        def _edge(j):
            cur = lax.rem(j, NBUF)
            pltpu.make_async_copy(utab.at[sidx.at[j]], rows.at[cur], gsem).wait()

            @pl.when(j + NBUF - 1 < ch_n)
            def _prefetch():
                pltpu.async_copy(utab.at[sidx.at[j + NBUF - 1]],
                                 rows.at[lax.rem(j + NBUF - 1, NBUF)], gsem)

            pltpu.sync_copy(rows.at[cur], acc.at[didx.at[j]], add=True)

    _prologue(u0_hbm)
    _zero_stripe()
    plsc.subcore_barrier()
    _phase(u0_hbm)
    _prologue(u1_hbm)
    plsc.subcore_barrier()
    pltpu.sync_copy(acc.at[pl.ds(s * STRIPE, STRIPE)],
                    out_hbm.at[c, 0, pl.ds(s * STRIPE, STRIPE)])
    _zero_stripe()
    plsc.subcore_barrier()
    _phase(u1_hbm)
    plsc.subcore_barrier()
    pltpu.sync_copy(acc.at[pl.ds(s * STRIPE, STRIPE)],
                    out_hbm.at[c, 1, pl.ds(s * STRIPE, STRIPE)])


_sc_msg = pl.kernel(
    _sc_msg_body,
    out_type=jax.ShapeDtypeStruct((NC, 2, NROW, D2), _f32),
    mesh=_mesh,
    compiler_params=pltpu.CompilerParams(
        needs_layout_passes=False, use_tc_tiling_on_sc=False),
    scratch_types=[
        pltpu.VMEM((CHMAX, K), jnp.int32),
        pltpu.VMEM((CHMAX, K), jnp.int32),
        pltpu.VMEM((NBUF, K, D2), _f32),
        pltpu.VMEM((K, D2), _f32),
        pltpu.VMEM_SHARED((NROW, D2), _f32),
        pltpu.SemaphoreType.DMA,
        pltpu.SemaphoreType.DMA,
    ],
)


def _bn_tc(x, g, b, eps=1e-5):
    mu = jnp.mean(x, axis=0, keepdims=True)
    xc = x - mu
    var = jnp.mean(xc * xc, axis=0, keepdims=True)
    return xc * lax.rsqrt(var + eps) * g + b


def _matmul_wt(h, w):
    return lax.dot_general(h, w, (((1,), (1,)), ((), ())),
                           preferred_element_type=_f32)


def _dinv_body(parts_ref, out_ref):
    deg = jnp.sum(parts_ref[...], axis=0, keepdims=True)
    deg_c = jnp.maximum(deg, 1e-12)
    out_ref[...] = jnp.where(deg > 0, lax.rsqrt(deg_c), 0.0)


def _split_cols(u, out_ref):
    out_ref[0, :, :] = u[:, :D2]
    out_ref[1, :, :] = u[:, D2:]


def _merge_s(s_ref):
    return jnp.concatenate(
        [s_ref[0, 0, :N, :] + s_ref[1, 0, :N, :],
         s_ref[0, 1, :N, :] + s_ref[1, 1, :N, :]], axis=1)


def _pre_body(x_ref, w_ref, ewr, ebr, g0, b0, gw, bw, w0, dinv_ref, out_ref):
    h = _bn_tc(x_ref[...], g0[...], b0[...])
    ew = w_ref[...] * ewr[...] + ebr[...]
    ew = _bn_tc(ew, gw[...], bw[...])
    h = jnp.maximum(h + ew, 0.0)
    u = _matmul_wt(h, w0[...])
    _split_cols(u * dinv_ref[...][:N, :], out_ref)


def _mid_body(s_ref, dinv_ref, bprev, g, bb, w, out_ref):
    dinv = dinv_ref[...][:N, :]
    agg = _merge_s(s_ref) * dinv + bprev[...]
    h = jnp.maximum(_bn_tc(agg, g[...], bb[...]), 0.0)
    _split_cols(_matmul_wt(h, w[...]) * dinv, out_ref)


def _final_body(s_ref, dinv_ref, b5, batch_ref, lw, lb, out_ref):
    dinv = dinv_ref[...][:N, :]
    agg = _merge_s(s_ref) * dinv + b5[...]
    onehot = (batch_ref[...] == lax.broadcasted_iota(
        jnp.int32, (1, G), 1)).astype(_f32)
    sums = lax.dot_general(onehot, agg, (((0,), (0,)), ((), ())),
                           preferred_element_type=_f32)
    cnt = lax.dot_general(onehot, jnp.ones((N, 1), _f32),
                          (((0,), (0,)), ((), ())),
                          preferred_element_type=_f32)
    red = jnp.sum(sums * lw[...], axis=1, keepdims=True)
    out_ref[...] = red / jnp.maximum(cnt, 1.0) + lb[0, 0]


def _tc(body, out_shape, *args):
    return pl.pallas_call(
        body, out_shape=jax.ShapeDtypeStruct(out_shape, _f32))(*args)


def kernel(x, edge_index, batch, weights, embed_W, embed_b, bn0_g, bn0_b,
           bnw_g, bnw_b, conv_W, conv_b, bn_g, bn_b, lin_W, lin_b):
    i32 = jnp.int32
    loop_idx = jnp.arange(N, dtype=i32)
    src = jnp.concatenate(
        [edge_index[0].astype(i32), loop_idx,
         jnp.zeros((PAD + CHMAX * K,), i32)]).reshape(NCH + CHMAX, K)
    dst = jnp.concatenate(
        [edge_index[1].astype(i32), loop_idx,
         jnp.full((PAD + CHMAX * K,), SCRAP, i32)]).reshape(NCH + CHMAX, K)

    deg_parts = _sc_deg(dst)
    dinv_col = _tc(_dinv_body, (1, NROW), deg_parts).reshape(NROW, 1)

    row = lambda v: v.reshape(1, D).astype(_f32)
    u = _tc(_pre_body, (2, N, D2),
            x.astype(_f32), weights.reshape(N, 1).astype(_f32),
            row(embed_W), row(embed_b), row(bn0_g), row(bn0_b),
            row(bnw_g), row(bnw_b), conv_W[0], dinv_col)

    for i in range(1, NLAYERS):
        sparts = _sc_msg(u[0], u[1], src, dst)
        u = _tc(_mid_body, (2, N, D2),
                sparts, dinv_col, row(conv_b[i - 1]),
                row(bn_g[i - 1]), row(bn_b[i - 1]), conv_W[i])

    sparts = _sc_msg(u[0], u[1], src, dst)
    out = _tc(_final_body, (G, 1),
              sparts, dinv_col, row(conv_b[NLAYERS - 1]),
              batch.reshape(N, 1).astype(i32), lin_W.astype(_f32),
              lin_b.reshape(1, 1).astype(_f32))
    return out.reshape(G)
